# baseline (speedup 1.0000x reference)
import jax
import jax.numpy as jnp
from jax import lax
from jax.experimental import pallas as pl
from jax.experimental.pallas import tpu as pltpu

_K = 1024
_D = 64
_N = 32768
_CH = 4096
_COMMIT = 0.25


def _vq_body(x_ref, wt_ref, wt2_ref, q_ref, loss_ref, perp_ref, lsum, hist):
    b = pl.program_id(0)
    j = pl.program_id(1)
    nb = pl.num_programs(0)
    nj = pl.num_programs(1)

    xt = x_ref[0, :, pl.ds(j * _CH, _CH)]
    x = jnp.transpose(xt)
    wt = wt_ref[...]
    wt2 = wt2_ref[...]

    s2 = lax.dot_general(x, wt2, (((1,), (0,)), ((), ())),
                         preferred_element_type=jnp.float32)
    xn = jnp.sum(x * x, axis=1, keepdims=True)
    wn = jnp.sum(wt * wt, axis=0, keepdims=True)
    d = (xn + wn) - s2

    m = jnp.min(d, axis=1, keepdims=True)
    iot = jnp.broadcast_to(
        lax.broadcasted_iota(jnp.int32, (1, _K), 1).astype(jnp.float32),
        d.shape)
    idx = jnp.min(jnp.where(d == m, iot, float(_K)), axis=1, keepdims=True)
    onehot = (iot == idx).astype(jnp.float32)

    qt = lax.dot_general(wt, onehot, (((1,), (1,)), ((), ())),
                         preferred_element_type=jnp.float32)
    q_ref[0] = qt

    @pl.when(jnp.logical_and(b == 0, j == 0))
    def _init():
        lsum[...] = jnp.zeros_like(lsum)
        hist[...] = jnp.zeros_like(hist)

    ones_row = jnp.ones((1, _CH), jnp.float32)
    lsum[...] += jnp.sum(m, keepdims=True).reshape(1, 1)
    hist[...] += lax.dot_general(ones_row, onehot, (((1,), (0,)), ((), ())),
                                 preferred_element_type=jnp.float32)

    @pl.when(jnp.logical_and(b == nb - 1, j == nj - 1))
    def _finalize():
        mse = lsum[...] / float(_N * _D)
        loss_ref[...] = (1.0 + _COMMIT) * mse
        p = hist[...] / float(_N)
        ent = jnp.sum(p * jnp.log(p + 1e-10), axis=1, keepdims=True)
        perp_ref[...] = jnp.exp(-ent)


def kernel(inputs, embedding_weight):
    b, c, t, h, w = inputs.shape
    x3 = inputs.reshape(b, c, t * h * w)
    wt = embedding_weight.T
    wt2 = wt + wt

    nj = (t * h * w) // _CH
    qt, loss, perp = pl.pallas_call(
        _vq_body,
        grid=(b, nj),
        in_specs=[
            pl.BlockSpec((1, _D, t * h * w), lambda i, j: (i, 0, 0)),
            pl.BlockSpec((_D, _K), lambda i, j: (0, 0)),
            pl.BlockSpec((_D, _K), lambda i, j: (0, 0)),
        ],
        out_specs=[
            pl.BlockSpec((1, _D, _CH), lambda i, j: (i, 0, j)),
            pl.BlockSpec((1, 1), lambda i, j: (0, 0)),
            pl.BlockSpec((1, 1), lambda i, j: (0, 0)),
        ],
        out_shape=[
            jax.ShapeDtypeStruct((b, _D, t * h * w), jnp.float32),
            jax.ShapeDtypeStruct((1, 1), jnp.float32),
            jax.ShapeDtypeStruct((1, 1), jnp.float32),
        ],
        scratch_shapes=[
            pltpu.VMEM((1, 1), jnp.float32),
            pltpu.VMEM((1, _K), jnp.float32),
        ],
    )(x3, wt, wt2)

    quantized = qt.reshape(b, c, t, h, w)
    return quantized, loss[0, 0], perp[0, 0]


# speedup vs baseline: 1.1536x; 1.1536x over previous
import jax
import jax.numpy as jnp
from jax import lax
from jax.experimental import pallas as pl
from jax.experimental.pallas import tpu as pltpu

_K = 1024
_D = 64
_N = 32768
_CH = 4096
_COMMIT = 0.25


def _vq_body(x_ref, wt_ref, wt2_ref, q_ref, loss_ref, perp_ref, lsum, hist):
    b = pl.program_id(0)
    j = pl.program_id(1)
    nb = pl.num_programs(0)
    nj = pl.num_programs(1)

    x = x_ref[...]
    wt = wt_ref[...]
    wt2 = wt2_ref[...]

    s2 = lax.dot_general(x, wt2, (((1,), (0,)), ((), ())),
                         preferred_element_type=jnp.float32)
    xn = jnp.sum(x * x, axis=1, keepdims=True)
    wn = jnp.sum(wt * wt, axis=0, keepdims=True)
    d = (xn + wn) - s2

    m = jnp.min(d, axis=1, keepdims=True)
    iot = jnp.broadcast_to(
        lax.broadcasted_iota(jnp.int32, (1, _K), 1).astype(jnp.float32),
        d.shape)
    idx = jnp.min(jnp.where(d == m, iot, float(_K)), axis=1, keepdims=True)
    onehot = (iot == idx).astype(jnp.float32)

    qt = lax.dot_general(wt, onehot, (((1,), (1,)), ((), ())),
                         preferred_element_type=jnp.float32)
    q_ref[0] = qt

    @pl.when(jnp.logical_and(b == 0, j == 0))
    def _init():
        lsum[...] = jnp.zeros_like(lsum)
        hist[...] = jnp.zeros_like(hist)

    ones_row = jnp.ones((1, _CH), jnp.float32)
    lsum[...] += jnp.sum(m, keepdims=True).reshape(1, 1)
    hist[...] += lax.dot_general(ones_row, onehot, (((1,), (0,)), ((), ())),
                                 preferred_element_type=jnp.float32)

    @pl.when(jnp.logical_and(b == nb - 1, j == nj - 1))
    def _finalize():
        mse = lsum[...] / float(_N * _D)
        loss_ref[...] = (1.0 + _COMMIT) * mse
        p = hist[...] / float(_N)
        ent = jnp.sum(p * jnp.log(p + 1e-10), axis=1, keepdims=True)
        perp_ref[...] = jnp.exp(-ent)


def kernel(inputs, embedding_weight):
    b, c, t, h, w = inputs.shape
    flat = jnp.transpose(inputs, (0, 2, 3, 4, 1)).reshape(-1, c)
    wt = embedding_weight.T
    wt2 = wt + wt

    nj = (t * h * w) // _CH
    qt, loss, perp = pl.pallas_call(
        _vq_body,
        grid=(b, nj),
        in_specs=[
            pl.BlockSpec((_CH, _D), lambda i, j, _nj=nj: (i * _nj + j, 0)),
            pl.BlockSpec((_D, _K), lambda i, j: (0, 0)),
            pl.BlockSpec((_D, _K), lambda i, j: (0, 0)),
        ],
        out_specs=[
            pl.BlockSpec((1, _D, _CH), lambda i, j: (i, 0, j)),
            pl.BlockSpec((1, 1), lambda i, j: (0, 0)),
            pl.BlockSpec((1, 1), lambda i, j: (0, 0)),
        ],
        out_shape=[
            jax.ShapeDtypeStruct((b, _D, t * h * w), jnp.float32),
            jax.ShapeDtypeStruct((1, 1), jnp.float32),
            jax.ShapeDtypeStruct((1, 1), jnp.float32),
        ],
        scratch_shapes=[
            pltpu.VMEM((1, 1), jnp.float32),
            pltpu.VMEM((1, _K), jnp.float32),
        ],
    )(flat, wt, wt2)

    quantized = qt.reshape(b, c, t, h, w)
    return quantized, loss[0, 0], perp[0, 0]
